# Initial kernel scaffold; baseline (speedup 1.0000x reference)
import jax
import jax.numpy as jnp
from jax.experimental import pallas as pl


def kernel(params, ere_ids, stmt_ids, head_idx, tail_idx, cand_idx):
    raise NotImplementedError("write your pallas kernel here")


# speedup vs baseline: 4.8305x; 4.8305x over previous
import functools

import jax
import jax.numpy as jnp
from jax import lax
from jax.experimental import pallas as pl
from jax.experimental.pallas import tpu as pltpu
from jax.experimental.pallas import tpu_sc as plsc

N_ERE = 10000
N_STMT = 160000
D = 128
N_CAND = 64

NC = 2
NS = 16
NW = NC * NS

_CH = 128
_NCHUNK = N_STMT // _CH
_NCHUNK_PAD = 1280
_SEG_PER_TILE = _NCHUNK_PAD // NS
_G_PER_W = _NCHUNK_PAD // NW
_ZS = 624
_ZTAIL = N_ERE - NS * _ZS

_f32 = jnp.float32
_bf16 = jnp.bfloat16


@functools.cache
def _sc_mesh():
    return plsc.VectorSubcoreMesh(core_axis_name="c", subcore_axis_name="s",
                                  num_cores=NC, num_subcores=NS)


def _bT(x, w):
    return lax.dot_general(x.astype(_bf16), w.astype(_bf16),
                           (((1,), (1,)), ((), ())),
                           preferred_element_type=_f32)


def _b0(a, b):
    return lax.dot_general(a.astype(_bf16), b.astype(_bf16),
                           (((0,), (0,)), ((), ())),
                           preferred_element_type=_f32)


def _sc_segsum_body(pq, idx3, zeros, out_ht, acc, idx_v, rows_v):
    c = lax.axis_index("c")
    s = lax.axis_index("s")
    pltpu.sync_copy(zeros.at[pl.ds(s * _ZS, _ZS)],
                    acc.at[pl.ds(s * _ZS, _ZS)])

    @pl.when(s == NS - 1)
    def _():
        pltpu.sync_copy(zeros.at[pl.ds(NS * _ZS, _ZTAIL)],
                        acc.at[pl.ds(NS * _ZS, _ZTAIL)])

    base = s * _SEG_PER_TILE
    pltpu.sync_copy(idx3.at[c].at[pl.ds(base, _SEG_PER_TILE)], idx_v)
    plsc.subcore_barrier()

    def step(j, carry):
        @pl.when(base + j < _NCHUNK)
        def _():
            pltpu.sync_copy(pq.at[c].at[pl.ds((base + j) * _CH, _CH)], rows_v)
            pltpu.sync_copy(rows_v, acc.at[idx_v.at[j]], add=True)
        return carry

    lax.fori_loop(0, _SEG_PER_TILE, step, 0)
    plsc.subcore_barrier()
    pltpu.sync_copy(acc.at[pl.ds(s * _ZS, _ZS)],
                    out_ht.at[c].at[pl.ds(s * _ZS, _ZS)])

    @pl.when(s == NS - 1)
    def _():
        pltpu.sync_copy(acc.at[pl.ds(NS * _ZS, _ZTAIL)],
                        out_ht.at[c].at[pl.ds(NS * _ZS, _ZTAIL)])


def _sc_segsum(pq, idx3, zeros):
    out = pl.kernel(
        _sc_segsum_body,
        out_type=jax.ShapeDtypeStruct((NC, N_ERE, D), _f32),
        mesh=_sc_mesh(),
        scratch_types=[
            pltpu.VMEM_SHARED((N_ERE, D), _f32),
            pltpu.VMEM((_SEG_PER_TILE, _CH), jnp.int32),
            pltpu.VMEM((_CH, D), _f32),
        ],
    )(pq, idx3, zeros)
    return out[0], out[1]


def _sc_gather_body(tab_h, tab_t, head2, tail2, out_h, out_t,
                    ih_v, it_v, bh_v, bt_v, sem_h, sem_t):
    c = lax.axis_index("c")
    s = lax.axis_index("s")
    w = s * NC + c
    base = w * _G_PER_W
    pltpu.sync_copy(head2.at[pl.ds(base, _G_PER_W)], ih_v)
    pltpu.sync_copy(tail2.at[pl.ds(base, _G_PER_W)], it_v)

    def step(j, carry):
        r = base + j

        @pl.when(r < _NCHUNK)
        def _():
            ch = pltpu.async_copy(tab_h.at[ih_v.at[j]], bh_v, sem_h)
            ct = pltpu.async_copy(tab_t.at[it_v.at[j]], bt_v, sem_t)
            ch.wait()
            ct.wait()
            pltpu.sync_copy(bh_v, out_h.at[pl.ds(r * _CH, _CH)])
            pltpu.sync_copy(bt_v, out_t.at[pl.ds(r * _CH, _CH)])
        return carry

    lax.fori_loop(0, _G_PER_W, step, 0)


def _sc_gather(tab_h, tab_t, head2, tail2):
    return pl.kernel(
        _sc_gather_body,
        out_type=(jax.ShapeDtypeStruct((N_STMT, D), _f32),
                  jax.ShapeDtypeStruct((N_STMT, D), _f32)),
        mesh=_sc_mesh(),
        scratch_types=[
            pltpu.VMEM((_G_PER_W, _CH), jnp.int32),
            pltpu.VMEM((_G_PER_W, _CH), jnp.int32),
            pltpu.VMEM((_CH, D), _f32),
            pltpu.VMEM((_CH, D), _f32),
            pltpu.SemaphoreType.DMA,
            pltpu.SemaphoreType.DMA,
        ],
    )(tab_h, tab_t, head2, tail2)


def _sc_cand_body(stmt_h, gh2, gt2, cand, out_hc, out_gh, out_gt,
                  idx_v, buf_v, sem):
    c = lax.axis_index("c")
    s = lax.axis_index("s")

    @pl.when(jnp.logical_and(c == 0, s == 0))
    def _():
        pltpu.sync_copy(cand, idx_v)
        pltpu.async_copy(stmt_h.at[idx_v], buf_v, sem).wait()
        pltpu.sync_copy(buf_v, out_hc)
        pltpu.async_copy(gh2.at[idx_v], buf_v, sem).wait()
        pltpu.sync_copy(buf_v, out_gh)
        pltpu.async_copy(gt2.at[idx_v], buf_v, sem).wait()
        pltpu.sync_copy(buf_v, out_gt)


def _sc_cand(stmt_h, gh2, gt2, cand):
    return pl.kernel(
        _sc_cand_body,
        out_type=(jax.ShapeDtypeStruct((N_CAND, D), _f32),
                  jax.ShapeDtypeStruct((N_CAND, D), _f32),
                  jax.ShapeDtypeStruct((N_CAND, D), _f32)),
        mesh=_sc_mesh(),
        scratch_types=[
            pltpu.VMEM((N_CAND,), jnp.int32),
            pltpu.VMEM((N_CAND, D), _f32),
            pltpu.SemaphoreType.DMA,
        ],
    )(stmt_h, gh2, gt2, cand)


_STMT_BLK = 3200


def _tc_pq_body(x, wh, wty, wt, pq):
    xb = x[...]
    pq[0] = _bT(xb, wh[...]) + _bT(xb, wty[...])
    pq[1] = _bT(xb, wt[...])


def _tc_pq(x, wh, wty, wt):
    nb = N_STMT // _STMT_BLK
    row = pl.BlockSpec((_STMT_BLK, D), lambda i: (i, 0))
    wsp = pl.BlockSpec((D, D), lambda i: (0, 0))
    return pl.pallas_call(
        _tc_pq_body,
        grid=(nb,),
        in_specs=[row, wsp, wsp, wsp],
        out_specs=pl.BlockSpec((NC, _STMT_BLK, D), lambda i: (0, i, 0)),
        out_shape=jax.ShapeDtypeStruct((NC, N_STMT, D), _f32),
    )(x, wh, wty, wt)


_ERE_BLK = 1000


def _tc_ere_body(base, sh, st, wself, wgh, wgt, eo, gho, gto):
    e = jnp.tanh(_bT(base[...], wself[...]) + sh[...] + st[...])
    eo[...] = e
    gho[...] = _bT(e, wgh[...])
    gto[...] = _bT(e, wgt[...])


def _tc_ere(base, sh, st, wself, wgh, wgt):
    nb = N_ERE // _ERE_BLK
    row = pl.BlockSpec((_ERE_BLK, D), lambda i: (i, 0))
    wsp = pl.BlockSpec((D, D), lambda i: (0, 0))
    return pl.pallas_call(
        _tc_ere_body,
        grid=(nb,),
        in_specs=[row, row, row, wsp, wsp, wsp],
        out_specs=[row, row, row],
        out_shape=[jax.ShapeDtypeStruct((N_ERE, D), _f32)] * 3,
    )(base, sh, st, wself, wgh, wgt)


def _tc_stmt_pq_body(x, gh, gt, w, wh2, wty2, wt2, o, pq):
    h = jnp.tanh(_bT(x[...], w[...]) + gh[...] + gt[...])
    o[...] = h
    pq[0] = _bT(h, wh2[...]) + _bT(h, wty2[...])
    pq[1] = _bT(h, wt2[...])


def _tc_stmt_pq(x, gh, gt, w, wh2, wty2, wt2):
    nb = N_STMT // _STMT_BLK
    row = pl.BlockSpec((_STMT_BLK, D), lambda i: (i, 0))
    wsp = pl.BlockSpec((D, D), lambda i: (0, 0))
    return pl.pallas_call(
        _tc_stmt_pq_body,
        grid=(nb,),
        in_specs=[row, row, row, wsp, wsp, wsp, wsp],
        out_specs=[row, pl.BlockSpec((NC, _STMT_BLK, D), lambda i: (0, i, 0))],
        out_shape=[jax.ShapeDtypeStruct((N_STMT, D), _f32),
                   jax.ShapeDtypeStruct((NC, N_STMT, D), _f32)],
    )(x, gh, gt, w, wh2, wty2, wt2)


def _tc_flash_body(x, gh, gt, w2, hc, ghc, gtc, wss,
                   ctx_o, att_o, att_s, m_s, l_s, ctx_s):
    i = pl.program_id(0)

    @pl.when(i == 0)
    def _():
        att = jnp.tanh(_bT(hc[...], w2[...]) + ghc[...] + gtc[...])
        att_s[...] = att
        att_o[...] = att
        m_s[...] = jnp.full((N_CAND,), -1e30, _f32)
        l_s[...] = jnp.zeros((N_CAND,), _f32)
        ctx_s[...] = jnp.zeros((N_CAND, D), _f32)

    h2 = jnp.tanh(_bT(x[...], w2[...]) + gh[...] + gt[...])
    v = _bT(h2, wss[...])
    sblk = _bT(v, att_s[...])
    m_old = m_s[...]
    m_new = jnp.maximum(m_old, jnp.max(sblk, axis=0))
    corr = jnp.exp(m_old - m_new)
    e = jnp.exp(sblk - m_new[None, :])
    l_s[...] = l_s[...] * corr + jnp.sum(e, axis=0)
    ctx_s[...] = ctx_s[...] * corr[:, None] + _b0(e, h2)
    m_s[...] = m_new

    @pl.when(i == pl.num_programs(0) - 1)
    def _():
        ctx_o[...] = ctx_s[...] / l_s[...][:, None]


def _tc_flash(x, gh, gt, w2, hc, ghc, gtc, wss):
    nb = N_STMT // _STMT_BLK
    row = pl.BlockSpec((_STMT_BLK, D), lambda i: (i, 0))
    full = lambda r: pl.BlockSpec((r, D), lambda i: (0, 0))
    return pl.pallas_call(
        _tc_flash_body,
        grid=(nb,),
        in_specs=[row, row, row, full(D), full(N_CAND), full(N_CAND),
                  full(N_CAND), full(D)],
        out_specs=[full(N_CAND), full(N_CAND)],
        out_shape=[jax.ShapeDtypeStruct((N_CAND, D), _f32)] * 2,
        scratch_shapes=[
            pltpu.VMEM((N_CAND, D), _f32),
            pltpu.VMEM((N_CAND,), _f32),
            pltpu.VMEM((N_CAND,), _f32),
            pltpu.VMEM((N_CAND, D), _f32),
        ],
    )(x, gh, gt, w2, hc, ghc, gtc, wss)


def _tc_final_body(ere2, att, ctx_ss, wes, wa, wcoh, o):
    e2 = ere2[...]
    v_es = _bT(e2, wes[...])
    s_es = _bT(v_es, att[...])
    mx = jnp.max(s_es, axis=0)
    ex = jnp.exp(s_es - mx[None, :])
    w_es = ex / jnp.sum(ex, axis=0)[None, :]
    ctx_es = _b0(w_es, e2)
    cat = jnp.concatenate([att[...], ctx_ss[...], ctx_es], axis=-1)
    av = jnp.tanh(_bT(cat, wa[...]))
    avb = av.astype(_bf16).astype(_f32)
    wb = wcoh[...].astype(_bf16).astype(_f32)
    o[...] = jnp.sum(avb * wb, axis=1)


def _tc_final(ere2, att, ctx_ss, wes, wa, wcoh):
    full = lambda shape: pl.BlockSpec(shape, lambda: tuple(0 for _ in shape))
    return pl.pallas_call(
        _tc_final_body,
        in_specs=[full((N_ERE, D)), full((N_CAND, D)), full((N_CAND, D)),
                  full((D, D)), full((D, 3 * D)), full((1, D))],
        out_specs=full((N_CAND,)),
        out_shape=jax.ShapeDtypeStruct((N_CAND,), _f32),
    )(ere2, att, ctx_ss, wes, wa, wcoh)


def kernel(params, ere_ids, stmt_ids, head_idx, tail_idx, cand_idx):
    p = params
    W = lambda n: p[n][0]
    stmt_e = p['stmt_table']
    ere_e = p['ere_table']
    pad = _NCHUNK_PAD * _CH - N_STMT
    head2 = jnp.pad(head_idx.astype(jnp.int32), (0, pad)).reshape(
        _NCHUNK_PAD, _CH)
    tail2 = jnp.pad(tail_idx.astype(jnp.int32), (0, pad)).reshape(
        _NCHUNK_PAD, _CH)
    idx3 = jnp.stack([head2, tail2])
    cand = cand_idx.astype(jnp.int32)
    zeros = jnp.zeros((N_ERE, D), _f32)

    pq1 = _tc_pq(stmt_e, W('head_adj_stmt_init'), W('type_adj_stmt_init'),
                 W('tail_adj_stmt_init'))
    s1h, s1t = _sc_segsum(pq1, idx3, zeros)
    ere_h, g1h, g1t = _tc_ere(ere_e, s1h, s1t, W('ere_init'),
                              W('head_adj_ere_init'), W('tail_adj_ere_init'))
    gh1, gt1 = _sc_gather(g1h, g1t, head2, tail2)

    stmt_h, pq2 = _tc_stmt_pq(stmt_e, gh1, gt1, W('stmt_init'),
                              W('head_adj_stmt'), W('type_adj_stmt'),
                              W('tail_adj_stmt'))
    s2h, s2t = _sc_segsum(pq2, idx3, zeros)
    ere_h2, g2h, g2t = _tc_ere(ere_h, s2h, s2t, W('ere'),
                               W('head_adj_ere'), W('tail_adj_ere'))
    gh2, gt2 = _sc_gather(g2h, g2t, head2, tail2)

    hc, ghc, gtc = _sc_cand(stmt_h, gh2, gt2, cand)
    ctx_ss, att = _tc_flash(stmt_h, gh2, gt2, W('stmt'), hc, ghc, gtc,
                            W('att_bilinear_ss'))
    return _tc_final(ere_h2, att, ctx_ss, W('att_bilinear_es'),
                     W('att_linear'), W('coherence_linear'))
